# baseline (speedup 1.0000x reference)
import functools

import jax
import jax.numpy as jnp
from jax import lax
from jax.experimental import pallas as pl
from jax.experimental.pallas import tpu as pltpu
from jax.experimental.pallas import tpu_sc as plsc

B = 32
T = 2048
F = 128
D = 64
ROWS_PER_WORKER = 8
TB = 128


def _sc_gather_kernel(table0, table1, idx0, idx1, e0_out, e1_out,
                      idx_v, rows_v, sem):
    nc = 2
    wid = lax.axis_index("s") * nc + lax.axis_index("c")
    nchunks = B // ROWS_PER_WORKER

    @pl.when(wid < nchunks)
    def _():
        base = wid * ROWS_PER_WORKER
        pltpu.sync_copy(idx0.at[pl.ds(base, ROWS_PER_WORKER)], idx_v)
        pltpu.async_copy(table0.at[idx_v], rows_v, sem).wait()
        pltpu.sync_copy(rows_v, e0_out.at[pl.ds(base, ROWS_PER_WORKER)])

    @pl.when((wid >= nchunks) & (wid < 2 * nchunks))
    def _():
        base = (wid - nchunks) * ROWS_PER_WORKER
        pltpu.sync_copy(idx1.at[pl.ds(base, ROWS_PER_WORKER)], idx_v)
        pltpu.async_copy(table1.at[idx_v], rows_v, sem).wait()
        pltpu.sync_copy(rows_v, e1_out.at[pl.ds(base, ROWS_PER_WORKER)])


@functools.partial(jax.jit, static_argnums=())
def _sc_gather(table0, table1, idx0, idx1):
    mesh = plsc.VectorSubcoreMesh(core_axis_name="c", subcore_axis_name="s")
    return pl.kernel(
        _sc_gather_kernel,
        out_type=(
            jax.ShapeDtypeStruct((B, D), jnp.float32),
            jax.ShapeDtypeStruct((B, D), jnp.float32),
        ),
        mesh=mesh,
        scratch_types=[
            pltpu.VMEM((ROWS_PER_WORKER,), jnp.int32),
            pltpu.VMEM((ROWS_PER_WORKER, D), jnp.float32),
            pltpu.SemaphoreType.DMA,
        ],
        compiler_params=pltpu.CompilerParams(use_tc_tiling_on_sc=False),
    )(table0, table1, idx0, idx1)


def _tc_concat_kernel(spec_ref, e0_ref, e1_ref, out_ref):
    out_ref[:, :, 0:F] = spec_ref[...]
    meta = jnp.concatenate([e0_ref[...], e1_ref[...]], axis=-1)
    out_ref[:, :, F:F + 2 * D] = jnp.broadcast_to(meta, (B, TB, 2 * D))


def _tc_concat(spectrogram, e0, e1):
    grid = (T // TB,)
    return pl.pallas_call(
        _tc_concat_kernel,
        grid=grid,
        in_specs=[
            pl.BlockSpec((B, TB, F), lambda t: (0, t, 0)),
            pl.BlockSpec((B, 1, D), lambda t: (0, 0, 0)),
            pl.BlockSpec((B, 1, D), lambda t: (0, 0, 0)),
        ],
        out_specs=pl.BlockSpec((B, TB, F + 2 * D), lambda t: (0, t, 0)),
        out_shape=jax.ShapeDtypeStruct((B, T, F + 2 * D), jnp.float32),
        compiler_params=pltpu.CompilerParams(
            dimension_semantics=("arbitrary",),
        ),
    )(spectrogram, e0.reshape(B, 1, D), e1.reshape(B, 1, D))


def kernel(spectrogram, seq_metadata, table0, table1):
    idx0 = seq_metadata[:, 0].astype(jnp.int32)
    idx1 = seq_metadata[:, 1].astype(jnp.int32)
    e0, e1 = _sc_gather(table0, table1, idx0, idx1)
    return _tc_concat(spectrogram, e0, e1)


# speedup vs baseline: 1.0699x; 1.0699x over previous
import functools

import jax
import jax.numpy as jnp
from jax import lax
from jax.experimental import pallas as pl
from jax.experimental.pallas import tpu as pltpu
from jax.experimental.pallas import tpu_sc as plsc

B = 32
T = 2048
F = 128
D = 64
ROWS_PER_WORKER = 8
TB = 512


def _sc_gather_kernel(table0, table1, idx0, idx1, e0_out, e1_out,
                      idx_v, rows_v, sem):
    nc = 2
    wid = lax.axis_index("s") * nc + lax.axis_index("c")
    nchunks = B // ROWS_PER_WORKER

    @pl.when(wid < nchunks)
    def _():
        base = wid * ROWS_PER_WORKER
        pltpu.sync_copy(idx0.at[pl.ds(base, ROWS_PER_WORKER)], idx_v)
        pltpu.async_copy(table0.at[idx_v], rows_v, sem).wait()
        pltpu.sync_copy(rows_v, e0_out.at[pl.ds(base, ROWS_PER_WORKER)])

    @pl.when((wid >= nchunks) & (wid < 2 * nchunks))
    def _():
        base = (wid - nchunks) * ROWS_PER_WORKER
        pltpu.sync_copy(idx1.at[pl.ds(base, ROWS_PER_WORKER)], idx_v)
        pltpu.async_copy(table1.at[idx_v], rows_v, sem).wait()
        pltpu.sync_copy(rows_v, e1_out.at[pl.ds(base, ROWS_PER_WORKER)])


@functools.partial(jax.jit, static_argnums=())
def _sc_gather(table0, table1, idx0, idx1):
    mesh = plsc.VectorSubcoreMesh(core_axis_name="c", subcore_axis_name="s")
    return pl.kernel(
        _sc_gather_kernel,
        out_type=(
            jax.ShapeDtypeStruct((B, D), jnp.float32),
            jax.ShapeDtypeStruct((B, D), jnp.float32),
        ),
        mesh=mesh,
        scratch_types=[
            pltpu.VMEM((ROWS_PER_WORKER,), jnp.int32),
            pltpu.VMEM((ROWS_PER_WORKER, D), jnp.float32),
            pltpu.SemaphoreType.DMA,
        ],
        compiler_params=pltpu.CompilerParams(use_tc_tiling_on_sc=False),
    )(table0, table1, idx0, idx1)


def _tc_concat_kernel(spec_ref, e0_ref, e1_ref, out_ref):
    out_ref[:, :, 0:F] = spec_ref[...]
    meta = jnp.concatenate([e0_ref[...], e1_ref[...]], axis=-1)
    out_ref[:, :, F:F + 2 * D] = jnp.broadcast_to(meta, (B, TB, 2 * D))


def _tc_concat(spectrogram, e0, e1):
    grid = (T // TB,)
    return pl.pallas_call(
        _tc_concat_kernel,
        grid=grid,
        in_specs=[
            pl.BlockSpec((B, TB, F), lambda t: (0, t, 0)),
            pl.BlockSpec((B, 1, D), lambda t: (0, 0, 0)),
            pl.BlockSpec((B, 1, D), lambda t: (0, 0, 0)),
        ],
        out_specs=pl.BlockSpec((B, TB, F + 2 * D), lambda t: (0, t, 0)),
        out_shape=jax.ShapeDtypeStruct((B, T, F + 2 * D), jnp.float32),
        compiler_params=pltpu.CompilerParams(
            dimension_semantics=("arbitrary",),
        ),
    )(spectrogram, e0.reshape(B, 1, D), e1.reshape(B, 1, D))


def kernel(spectrogram, seq_metadata, table0, table1):
    idx0 = seq_metadata[:, 0].astype(jnp.int32)
    idx1 = seq_metadata[:, 1].astype(jnp.int32)
    e0, e1 = _sc_gather(table0, table1, idx0, idx1)
    return _tc_concat(spectrogram, e0, e1)
